# Initial kernel scaffold; baseline (speedup 1.0000x reference)
import jax
import jax.numpy as jnp
from jax.experimental import pallas as pl


def kernel(x, features, edge_index, edge_weight, W1, W2):
    raise NotImplementedError("write your pallas kernel here")


# speedup vs baseline: 3.8339x; 3.8339x over previous
import functools

import jax
import jax.numpy as jnp
from jax import lax
from jax.experimental import pallas as pl
from jax.experimental.pallas import tpu as pltpu
from jax.experimental.pallas import tpu_sc as plsc

N_CORES = 2
N_SUB = 16
N_WORKERS = N_CORES * N_SUB
CHUNK = 128


def _mm_body(h_ref, w_ref, o_ref):
    o_ref[...] = jnp.dot(h_ref[...], w_ref[...],
                         preferred_element_type=jnp.float32)


def _tc_matmul(h, w, blk):
    n, d = h.shape
    return pl.pallas_call(
        _mm_body,
        grid=(n // blk,),
        in_specs=[
            pl.BlockSpec((blk, d), lambda i: (i, 0)),
            pl.BlockSpec((d, d), lambda i: (0, 0)),
        ],
        out_specs=pl.BlockSpec((blk, d), lambda i: (i, 0)),
        out_shape=jax.ShapeDtypeStruct((n, d), jnp.float32),
    )(h, w)


def _relu_mm_body(p_ref, w_ref, o_ref):
    h = jnp.maximum(p_ref[0] + p_ref[1], 0.0)
    o_ref[...] = jnp.dot(h, w_ref[...], preferred_element_type=jnp.float32)


def _tc_relu_combine_matmul(partials, w, blk):
    _, n, d = partials.shape
    return pl.pallas_call(
        _relu_mm_body,
        grid=(n // blk,),
        in_specs=[
            pl.BlockSpec((2, blk, d), lambda i: (0, i, 0)),
            pl.BlockSpec((d, d), lambda i: (0, 0)),
        ],
        out_specs=pl.BlockSpec((blk, d), lambda i: (i, 0)),
        out_shape=jax.ShapeDtypeStruct((n, d), jnp.float32),
    )(partials, w)


def _add_body(p_ref, o_ref):
    o_ref[...] = p_ref[0] + p_ref[1]


def _tc_combine(partials, blk):
    _, n, d = partials.shape
    return pl.pallas_call(
        _add_body,
        grid=(n // blk,),
        in_specs=[pl.BlockSpec((2, blk, d), lambda i: (0, i, 0))],
        out_specs=pl.BlockSpec((blk, d), lambda i: (i, 0)),
        out_shape=jax.ShapeDtypeStruct((n, d), jnp.float32),
    )(partials)


def _make_edge_pass(n_pad, d_feat, n_chunks):
    rows_per_sub = n_pad // N_SUB
    assert rows_per_sub % 8 == 0
    mesh = plsc.VectorSubcoreMesh(core_axis_name="c", subcore_axis_name="s")

    @functools.partial(
        pl.kernel,
        mesh=mesh,
        out_type=jax.ShapeDtypeStruct((N_CORES, n_pad, d_feat),
                                      jnp.float32),
        scratch_types=[
            pltpu.VMEM((n_chunks, CHUNK), jnp.int32),
            pltpu.VMEM((n_chunks, CHUNK), jnp.int32),
            pltpu.VMEM((n_chunks, CHUNK), jnp.float32),
            pltpu.VMEM((CHUNK, d_feat), jnp.float32),
            pltpu.VMEM_SHARED((n_pad, d_feat), jnp.float32),
        ],
    )
    def edge_pass(sup_hbm, src_hbm, dst_hbm, ew_hbm, out_hbm,
                  src_v, dst_v, ew_v, gbuf, acc):
        c = lax.axis_index("c")
        s = lax.axis_index("s")
        wid = s * N_CORES + c

        pltpu.sync_copy(src_hbm.at[wid], src_v)
        pltpu.sync_copy(dst_hbm.at[wid], dst_v)
        pltpu.sync_copy(ew_hbm.at[wid], ew_v)

        zeros16 = jnp.zeros((16,), jnp.float32)

        def zero_row(r, _):
            for v in range(d_feat // 16):
                gbuf[r, pl.ds(v * 16, 16)] = zeros16
            return 0

        lax.fori_loop(0, CHUNK, zero_row, 0)
        row0 = s * rows_per_sub
        off = 0
        while off < rows_per_sub:
            n = min(CHUNK, rows_per_sub - off)
            pltpu.sync_copy(gbuf.at[pl.ds(0, n)],
                            acc.at[pl.ds(row0 + off, n)])
            off += n
        plsc.subcore_barrier()

        def do_chunk(j, _):
            pltpu.sync_copy(sup_hbm.at[src_v.at[j]], gbuf)

            def scale_group(g, _):
                wv = ew_v[j, pl.ds(g * 16, 16)]
                for e in range(16):
                    w = wv[e]
                    k = g * 16 + e
                    for v in range(d_feat // 16):
                        sl = pl.ds(v * 16, 16)
                        gbuf[k, sl] = gbuf[k, sl] * w
                return 0

            lax.fori_loop(0, CHUNK // 16, scale_group, 0)
            pltpu.sync_copy(gbuf, acc.at[dst_v.at[j]], add=True)
            return 0

        lax.fori_loop(0, n_chunks, do_chunk, 0)
        plsc.subcore_barrier()

        pltpu.sync_copy(acc.at[pl.ds(row0, rows_per_sub)],
                        out_hbm.at[c, pl.ds(row0, rows_per_sub)])

    return edge_pass


def kernel(x, features, edge_index, edge_weight, W1, W2):
    n_nodes, d_feat = features.shape
    n_edges = edge_weight.shape[0]

    n_pad = -(-n_nodes // (N_SUB * 8)) * (N_SUB * 8)
    while n_pad % 128 != 0:
        n_pad += N_SUB * 8
    blk = n_pad // 79 if n_pad % 79 == 0 else 128
    if n_pad % blk != 0:
        blk = N_SUB * 8

    hx = jnp.take(features, x, axis=0)
    hx = jnp.pad(hx, ((0, n_pad - n_nodes), (0, 0)))

    per_worker = -(-n_edges // (N_WORKERS * CHUNK)) * CHUNK
    e_pad = per_worker * N_WORKERS
    n_chunks = per_worker // CHUNK
    pad = e_pad - n_edges
    src = jnp.pad(edge_index[0].astype(jnp.int32), (0, pad))
    src = src.reshape(N_WORKERS, n_chunks, CHUNK)
    dst = jnp.pad(edge_index[1].astype(jnp.int32), (0, pad))
    dst = dst.reshape(N_WORKERS, n_chunks, CHUNK)
    ew = jnp.pad(edge_weight, (0, pad)).reshape(N_WORKERS, n_chunks, CHUNK)

    edge_pass = _make_edge_pass(n_pad, d_feat, n_chunks)

    s1 = _tc_matmul(hx, W1, blk)
    p1 = edge_pass(s1, src, dst, ew)
    s2 = _tc_relu_combine_matmul(p1, W2, blk)
    p2 = edge_pass(s2, src, dst, ew)
    return _tc_combine(p2, blk)[:n_nodes]


# speedup vs baseline: 4.3537x; 1.1356x over previous
import functools

import jax
import jax.numpy as jnp
from jax import lax
from jax.experimental import pallas as pl
from jax.experimental.pallas import tpu as pltpu
from jax.experimental.pallas import tpu_sc as plsc

N_CORES = 2
N_SUB = 16
CHUNK = 128
NBUF = 3


def _mm_split_body(h_ref, w_ref, o_ref):
    j = pl.program_id(0)
    d = w_ref.shape[0]
    dh = d // 2
    w = jnp.where(j == 0, w_ref[:, :dh], w_ref[:, dh:])
    o_ref[0] = jnp.dot(h_ref[...], w, preferred_element_type=jnp.float32)


def _tc_matmul_split(h, w, blk):
    n, d = h.shape
    dh = d // 2
    return pl.pallas_call(
        _mm_split_body,
        grid=(2, n // blk),
        in_specs=[
            pl.BlockSpec((blk, d), lambda j, i: (i, 0)),
            pl.BlockSpec((d, d), lambda j, i: (0, 0)),
        ],
        out_specs=pl.BlockSpec((1, blk, dh), lambda j, i: (j, i, 0)),
        out_shape=jax.ShapeDtypeStruct((2, n, dh), jnp.float32),
    )(h, w)


def _relu_mm_split_body(p_ref, w_ref, o_ref):
    j = pl.program_id(0)
    d = w_ref.shape[0]
    dh = d // 2
    w = jnp.where(j == 0, w_ref[:, :dh], w_ref[:, dh:])
    h = jnp.concatenate(
        [jnp.maximum(p_ref[0], 0.0), jnp.maximum(p_ref[1], 0.0)], axis=1)
    o_ref[0] = jnp.dot(h, w, preferred_element_type=jnp.float32)


def _tc_relu_matmul_split(halves, w, blk):
    _, n, dh = halves.shape
    d = 2 * dh
    return pl.pallas_call(
        _relu_mm_split_body,
        grid=(2, n // blk),
        in_specs=[
            pl.BlockSpec((2, blk, dh), lambda j, i: (0, i, 0)),
            pl.BlockSpec((d, d), lambda j, i: (0, 0)),
        ],
        out_specs=pl.BlockSpec((1, blk, dh), lambda j, i: (j, i, 0)),
        out_shape=jax.ShapeDtypeStruct((2, n, dh), jnp.float32),
    )(halves, w)


def _make_edge_pass(n_pad, d_half, n_chunks):
    rows_per_sub = n_pad // N_SUB
    assert rows_per_sub % 8 == 0
    assert n_chunks % NBUF == 0
    mesh = plsc.VectorSubcoreMesh(core_axis_name="c", subcore_axis_name="s")

    @functools.partial(
        pl.kernel,
        mesh=mesh,
        compiler_params=pltpu.CompilerParams(use_tc_tiling_on_sc=False),
        out_type=jax.ShapeDtypeStruct((N_CORES, n_pad, d_half),
                                      jnp.float32),
        scratch_types=[
            pltpu.VMEM((n_chunks, CHUNK), jnp.int32),
            pltpu.VMEM((n_chunks, CHUNK), jnp.int32),
            pltpu.VMEM((n_chunks, CHUNK), jnp.float32),
            [pltpu.VMEM((CHUNK, d_half), jnp.float32)] * NBUF,
            pltpu.VMEM_SHARED((n_pad, d_half), jnp.float32),
            [pltpu.SemaphoreType.DMA] * 3,
            [pltpu.SemaphoreType.DMA] * NBUF,
            [pltpu.SemaphoreType.DMA] * NBUF,
        ],
    )
    def edge_pass(sup_hbm, src_hbm, dst_hbm, ew_hbm, out_hbm,
                  src_v, dst_v, ew_v, gb, acc, esem, gsem, ssem):
        c = lax.axis_index("c")
        s = lax.axis_index("s")

        e0 = pltpu.async_copy(src_hbm.at[s], src_v, esem[0])
        e1 = pltpu.async_copy(dst_hbm.at[s], dst_v, esem[1])
        e2 = pltpu.async_copy(ew_hbm.at[s], ew_v, esem[2])

        zeros16 = jnp.zeros((16,), jnp.float32)

        def zero_row(r, _):
            for v in range(d_half // 16):
                gb[0][r, pl.ds(v * 16, 16)] = zeros16
            return 0

        lax.fori_loop(0, CHUNK, zero_row, 0)
        row0 = s * rows_per_sub
        off = 0
        while off < rows_per_sub:
            n = min(CHUNK, rows_per_sub - off)
            pltpu.sync_copy(gb[0].at[pl.ds(0, n)],
                            acc.at[pl.ds(row0 + off, n)])
            off += n

        e0.wait()
        coff = c * n_pad

        def add_off(r, _):
            for v in range(CHUNK // 16):
                sl = pl.ds(v * 16, 16)
                src_v[r, sl] = src_v[r, sl] + coff
            return 0

        lax.fori_loop(0, n_chunks, add_off, 0)
        e1.wait()
        e2.wait()
        plsc.subcore_barrier()

        def issue_gather(a, b):
            pltpu.async_copy(sup_hbm.at[src_v.at[a]], gb[b], gsem[b])

        def wait_gather(a, b):
            pltpu.make_async_copy(sup_hbm.at[src_v.at[a]], gb[b],
                                  gsem[b]).wait()

        def issue_scatter(a, b):
            pltpu.async_copy(gb[b], acc.at[dst_v.at[a]], ssem[b], add=True)

        def wait_scatter(a, b):
            pltpu.make_async_copy(gb[b], acc.at[dst_v.at[a]],
                                  ssem[b]).wait()

        for b in range(NBUF - 1):
            issue_gather(b, b)

        def scale(b, j):
            def scale_group(g, _):
                wv = ew_v[j, pl.ds(g * 16, 16)]
                for e in range(16):
                    w = wv[e]
                    k = g * 16 + e
                    for v in range(d_half // 16):
                        sl = pl.ds(v * 16, 16)
                        gb[b][k, sl] = gb[b][k, sl] * w
                return 0

            lax.fori_loop(0, CHUNK // 16, scale_group, 0)

        def pipe_step(jj, _):
            for b in range(NBUF):
                a = jj * NBUF + b
                wait_gather(a, b)
                scale(b, a)
                issue_scatter(a, b)
                pb = (b + NBUF - 1) % NBUF

                @pl.when(a >= 1)
                def _():
                    wait_scatter(a - 1, pb)

                @pl.when(a + NBUF - 1 < n_chunks)
                def _():
                    issue_gather(a + NBUF - 1, pb)
            return 0

        lax.fori_loop(0, n_chunks // NBUF, pipe_step, 0)
        wait_scatter(n_chunks - 1, NBUF - 1)
        plsc.subcore_barrier()

        pltpu.sync_copy(acc.at[pl.ds(row0, rows_per_sub)],
                        out_hbm.at[c, pl.ds(row0, rows_per_sub)])

    return edge_pass


def kernel(x, features, edge_index, edge_weight, W1, W2):
    n_nodes, d_feat = features.shape
    d_half = d_feat // 2
    n_edges = edge_weight.shape[0]

    n_pad = -(-n_nodes // (N_SUB * 8)) * (N_SUB * 8)
    while n_pad % 128 != 0:
        n_pad += N_SUB * 8
    blk = n_pad // 79 if n_pad % 79 == 0 else 128
    if n_pad % blk != 0:
        blk = N_SUB * 8

    hx = jnp.take(features, x, axis=0)
    hx = jnp.pad(hx, ((0, n_pad - n_nodes), (0, 0)))

    per_sub = -(-n_edges // (N_SUB * CHUNK * NBUF)) * CHUNK * NBUF
    e_pad = per_sub * N_SUB
    n_chunks = per_sub // CHUNK
    pad = e_pad - n_edges
    src = jnp.pad(edge_index[0].astype(jnp.int32), (0, pad))
    src = src.reshape(N_SUB, n_chunks, CHUNK)
    dst = jnp.pad(edge_index[1].astype(jnp.int32), (0, pad))
    dst = dst.reshape(N_SUB, n_chunks, CHUNK)
    ew = jnp.pad(edge_weight, (0, pad)).reshape(N_SUB, n_chunks, CHUNK)

    edge_pass = _make_edge_pass(n_pad, d_half, n_chunks)

    s1 = _tc_matmul_split(hx, W1, blk).reshape(2 * n_pad, d_half)
    p1 = edge_pass(s1, src, dst, ew)
    s2 = _tc_relu_matmul_split(p1, W2, blk).reshape(2 * n_pad, d_half)
    p2 = edge_pass(s2, src, dst, ew)
    out = jnp.moveaxis(p2, 0, 1).reshape(n_pad, d_feat)
    return out[:n_nodes]
